# Initial kernel scaffold; baseline (speedup 1.0000x reference)
import jax
import jax.numpy as jnp
from jax.experimental import pallas as pl


def kernel(series):
    raise NotImplementedError("write your pallas kernel here")


# speedup vs baseline: 3.2966x; 3.2966x over previous
import functools

import jax
import jax.numpy as jnp
from jax import lax
from jax.experimental import pallas as pl
from jax.experimental.pallas import tpu as pltpu
from jax.experimental.pallas import tpu_sc as plsc

_PATCH = 16
_STRIDE = 8


def kernel(series):
    batch = series.shape[:-1]
    seq_len = series.shape[-1]
    rows = 1
    for d in batch:
        rows *= d
    n_chunks = seq_len // _STRIDE
    n_patches = (seq_len - _PATCH) // _STRIDE + 1

    x = series.reshape(rows, n_chunks, _STRIDE)

    num_workers = 32
    rows_per_w = rows // num_workers
    c_rows = 4
    n_steps = rows_per_w // c_rows

    mesh = plsc.VectorSubcoreMesh(core_axis_name="c", subcore_axis_name="s")

    @functools.partial(
        pl.kernel,
        out_type=jax.ShapeDtypeStruct((rows, n_patches, _PATCH), jnp.float32),
        mesh=mesh,
        scratch_types=[
            pltpu.VMEM((2, c_rows, n_patches, _PATCH), jnp.float32),
            pltpu.SemaphoreType.DMA((2,)),
            pltpu.SemaphoreType.DMA((2,)),
        ],
        compiler_params=pltpu.CompilerParams(use_tc_tiling_on_sc=False),
    )
    def patcher(in_hbm, out_hbm, obuf, gsem, wsem):
        wid = lax.axis_index("s") * 2 + lax.axis_index("c")
        base = wid * rows_per_w

        def start_gathers(step, slot):
            r0 = base + step * c_rows
            a = pltpu.make_async_copy(
                in_hbm.at[pl.ds(r0, c_rows), pl.ds(0, n_patches), :],
                obuf.at[slot, :, :, pl.ds(0, _STRIDE)],
                gsem.at[slot],
            )
            b = pltpu.make_async_copy(
                in_hbm.at[pl.ds(r0, c_rows), pl.ds(1, n_patches), :],
                obuf.at[slot, :, :, pl.ds(_STRIDE, _STRIDE)],
                gsem.at[slot],
            )
            a.start()
            b.start()
            return a, b

        def make_write(step, slot):
            r0 = base + step * c_rows
            return pltpu.make_async_copy(
                obuf.at[slot],
                out_hbm.at[pl.ds(r0, c_rows)],
                wsem.at[slot],
            )

        writes = [None, None]
        gathers = start_gathers(0, 0)
        for step in range(n_steps):
            slot = step % 2
            nslot = (step + 1) % 2
            ga, gb = gathers
            ga.wait()
            gb.wait()
            w = make_write(step, slot)
            w.start()
            if step + 1 < n_steps:
                if writes[nslot] is not None:
                    writes[nslot].wait()
                gathers = start_gathers(step + 1, nslot)
            writes[slot] = w
        writes[(n_steps - 2) % 2].wait()
        writes[(n_steps - 1) % 2].wait()

    out = patcher(x)
    return out.reshape(*batch, n_patches, _PATCH)


# speedup vs baseline: 4.9472x; 1.5007x over previous
import functools

import jax
import jax.numpy as jnp
from jax import lax
from jax.experimental import pallas as pl
from jax.experimental.pallas import tpu as pltpu
from jax.experimental.pallas import tpu_sc as plsc

_PATCH = 16
_STRIDE = 8


def kernel(series):
    batch = series.shape[:-1]
    seq_len = series.shape[-1]
    rows = 1
    for d in batch:
        rows *= d
    n_patches = (seq_len - _PATCH) // _STRIDE + 1

    x = series.reshape(rows, seq_len)

    num_workers = 32
    rows_per_w = rows // num_workers
    c_rows = 4
    n_steps = rows_per_w // c_rows

    mesh = plsc.VectorSubcoreMesh(core_axis_name="c", subcore_axis_name="s")

    @functools.partial(
        pl.kernel,
        out_type=jax.ShapeDtypeStruct((rows, n_patches, _PATCH), jnp.float32),
        mesh=mesh,
        scratch_types=[
            pltpu.VMEM((2, c_rows, seq_len), jnp.float32),
            pltpu.VMEM((2, c_rows, n_patches, _PATCH), jnp.float32),
            pltpu.SemaphoreType.DMA((2,)),
            pltpu.SemaphoreType.DMA((2,)),
        ],
        compiler_params=pltpu.CompilerParams(use_tc_tiling_on_sc=False),
    )
    def patcher(in_hbm, out_hbm, ibuf, obuf, gsem, wsem):
        wid = lax.axis_index("s") * 2 + lax.axis_index("c")
        base = wid * rows_per_w

        def start_gather(step, slot):
            r0 = base + step * c_rows
            cp = pltpu.make_async_copy(
                in_hbm.at[pl.ds(r0, c_rows)], ibuf.at[slot], gsem.at[slot]
            )
            cp.start()
            return cp

        def make_write(step, slot):
            r0 = base + step * c_rows
            return pltpu.make_async_copy(
                obuf.at[slot], out_hbm.at[pl.ds(r0, c_rows)], wsem.at[slot]
            )

        def repack(slot):
            for r in range(c_rows):
                src = ibuf.at[slot, r]
                dst = obuf.at[slot, r]

                @plsc.parallel_loop(0, n_patches, 1, unroll=8)
                def _(p):
                    dst[p, :] = src[pl.ds(p * _STRIDE, _PATCH)]

        writes = [None, None]
        g = start_gather(0, 0)
        for step in range(n_steps):
            slot = step % 2
            nslot = (step + 1) % 2
            g.wait()
            if step + 1 < n_steps:
                g = start_gather(step + 1, nslot)
            if writes[slot] is not None:
                writes[slot].wait()
            repack(slot)
            w = make_write(step, slot)
            w.start()
            writes[slot] = w
        writes[(n_steps - 2) % 2].wait()
        writes[(n_steps - 1) % 2].wait()

    out = patcher(x)
    return out.reshape(*batch, n_patches, _PATCH)
